# baseline (speedup 1.0000x reference)
import functools

import jax
import jax.numpy as jnp
from jax import lax
from jax.experimental import pallas as pl
from jax.experimental.pallas import tpu as pltpu
from jax.experimental.pallas import tpu_sc as plsc

NUM_CLASSES = 40
OUT_CHANNELS = 512
N = 100000

NUM_WORKERS = 32
CHUNK = 80
NUM_CHUNKS = N // CHUNK
MAX_T = (NUM_CHUNKS + NUM_WORKERS - 1) // NUM_WORKERS
SPAN = MAX_T * CHUNK
N_PAD = (NUM_CHUNKS - 1) * CHUNK + SPAN
LANES = 16
TABLE_ROWS = 48
VPR = OUT_CHANNELS // LANES


@functools.partial(
    pl.kernel,
    mesh=plsc.VectorSubcoreMesh(core_axis_name="c", subcore_axis_name="s"),
    out_type=jax.ShapeDtypeStruct((N, OUT_CHANNELS), jnp.float32),
    scratch_types=[
        pltpu.VMEM((TABLE_ROWS, OUT_CHANNELS), jnp.float32),
        pltpu.VMEM((SPAN,), jnp.int32),
        pltpu.VMEM((SPAN,), jnp.int32),
        pltpu.VMEM((CHUNK, OUT_CHANNELS), jnp.float32),
        pltpu.VMEM((CHUNK, OUT_CHANNELS), jnp.float32),
        pltpu.SemaphoreType.DMA,
        pltpu.SemaphoreType.DMA,
    ],
)
def _masked_gather(y_hbm, m_hbm, emb_hbm, out_hbm,
                   table_v, y_v, m_v, rows0, rows1, wsem0, wsem1):
    w = lax.axis_index("s") * 2 + lax.axis_index("c")
    nt = jnp.where(w < 2, MAX_T, MAX_T - 1)
    start_chunk = (MAX_T - 1) * w + jnp.minimum(w, 2)
    base = start_chunk * CHUNK

    pltpu.sync_copy(emb_hbm, table_v)
    pltpu.sync_copy(y_hbm.at[pl.ds(base, SPAN)], y_v)
    pltpu.sync_copy(m_hbm.at[pl.ds(base, SPAN)], m_v)

    rows = (rows0, rows1)
    wsem = (wsem0, wsem1)

    zeros_v = jnp.zeros((LANES,), jnp.float32)

    def fill_chunk(t, b):
        def group_body(g, carry):
            yv = y_v[pl.ds(t * CHUNK + g * LANES, LANES)]
            mv = m_v[pl.ds(t * CHUNK + g * LANES, LANES)]
            for r in range(LANES):
                @pl.when(mv[r] != 0)
                def _(r=r):
                    src_row = yv[r]
                    vals = [table_v[src_row, pl.ds(c * LANES, LANES)]
                            for c in range(VPR)]
                    for c in range(VPR):
                        rows[b][g * LANES + r, pl.ds(c * LANES, LANES)] = vals[c]
                @pl.when(mv[r] == 0)
                def _(r=r):
                    for c in range(VPR):
                        rows[b][g * LANES + r, pl.ds(c * LANES, LANES)] = zeros_v
            return carry
        lax.fori_loop(0, CHUNK // LANES, group_body, 0)

    def write_desc(t, b):
        dst = out_hbm.at[pl.ds(base + t * CHUNK, CHUNK)]
        return pltpu.make_async_copy(rows[b], dst, wsem[b])

    def pair_body(j, carry):
        for h in range(2):
            t = 2 * j + h
            @pl.when(jnp.logical_and(t >= 2, t - 2 < nt))
            def _(t=t, b=h):
                write_desc(t - 2, b).wait()
            @pl.when(t < nt)
            def _(t=t, b=h):
                fill_chunk(t, b)
                write_desc(t, b).start()
        return carry
    lax.fori_loop(0, MAX_T // 2 + 1, pair_body, 0)


def kernel(y, mask, emb):
    y32 = jnp.pad(y.astype(jnp.int32), (0, N_PAD - N))
    m32 = jnp.pad(mask.astype(jnp.int32), (0, N_PAD - N))
    emb2 = jnp.concatenate(
        [emb, jnp.zeros((TABLE_ROWS - NUM_CLASSES, OUT_CHANNELS), emb.dtype)],
        axis=0)
    return _masked_gather(y32, m32, emb2)


# speedup vs baseline: 1.8581x; 1.8581x over previous
import functools

import jax
import jax.numpy as jnp
from jax import lax
from jax.experimental import pallas as pl
from jax.experimental.pallas import tpu as pltpu
from jax.experimental.pallas import tpu_sc as plsc

NUM_CLASSES = 40
OUT_CHANNELS = 512
N = 100000

NUM_WORKERS = 32
CHUNK = 80
NUM_CHUNKS = N // CHUNK
MAX_T = (NUM_CHUNKS + NUM_WORKERS - 1) // NUM_WORKERS
SPAN = MAX_T * CHUNK
N_PAD = (NUM_CHUNKS - 1) * CHUNK + SPAN
LANES = 16
TABLE_ROWS = 48


@functools.partial(
    pl.kernel,
    mesh=plsc.VectorSubcoreMesh(core_axis_name="c", subcore_axis_name="s"),
    out_type=jax.ShapeDtypeStruct((N, OUT_CHANNELS), jnp.float32),
    scratch_types=[
        pltpu.VMEM((TABLE_ROWS, OUT_CHANNELS), jnp.float32),
        pltpu.VMEM((SPAN,), jnp.int32),
        pltpu.VMEM((SPAN,), jnp.int32),
        pltpu.VMEM((SPAN,), jnp.int32),
        pltpu.VMEM((CHUNK, OUT_CHANNELS), jnp.float32),
        pltpu.SemaphoreType.DMA,
        pltpu.SemaphoreType.DMA,
    ],
)
def _masked_gather(y_hbm, m_hbm, emb_hbm, out_hbm,
                   table_v, y_v, m_v, idx_v, dummy_v, sem0, sem1):
    w = lax.axis_index("s") * 2 + lax.axis_index("c")
    nt = jnp.where(w < 2, MAX_T, MAX_T - 1)
    start_chunk = (MAX_T - 1) * w + jnp.minimum(w, 2)
    base = start_chunk * CHUNK

    pltpu.sync_copy(emb_hbm, table_v)
    pltpu.sync_copy(y_hbm.at[pl.ds(base, SPAN)], y_v)
    pltpu.sync_copy(m_hbm.at[pl.ds(base, SPAN)], m_v)

    def sel_body(i, carry):
        sl = pl.ds(i * LANES, LANES)
        idx_v[sl] = jnp.where(m_v[sl] != 0, y_v[sl], NUM_CLASSES)
        return carry
    lax.fori_loop(0, SPAN // LANES, sel_body, 0)

    sems = (sem0, sem1)

    def issue_chunk(t, b):
        def group_body(g, carry):
            row0 = t * CHUNK + g * LANES
            idxv = idx_v[pl.ds(row0, LANES)]
            for r in range(LANES):
                pltpu.make_async_copy(
                    table_v.at[idxv[r]],
                    out_hbm.at[base + row0 + r],
                    sems[b]).start()
            return carry
        lax.fori_loop(0, CHUNK // LANES, group_body, 0)

    def drain_desc(t, b):
        return pltpu.make_async_copy(
            out_hbm.at[pl.ds(base + t * CHUNK, CHUNK)], dummy_v, sems[b])

    def pair_body(j, carry):
        for h in range(2):
            t = 2 * j + h
            @pl.when(jnp.logical_and(t >= 2, t - 2 < nt))
            def _(t=t, b=h):
                drain_desc(t - 2, b).wait()
            @pl.when(t < nt)
            def _(t=t, b=h):
                issue_chunk(t, b)
        return carry
    lax.fori_loop(0, MAX_T // 2 + 1, pair_body, 0)


def kernel(y, mask, emb):
    y32 = jnp.pad(y.astype(jnp.int32), (0, N_PAD - N))
    m32 = jnp.pad(mask.astype(jnp.int32), (0, N_PAD - N))
    emb2 = jnp.concatenate(
        [emb, jnp.zeros((TABLE_ROWS - NUM_CLASSES, OUT_CHANNELS), emb.dtype)],
        axis=0)
    return _masked_gather(y32, m32, emb2)


# speedup vs baseline: 1.8785x; 1.0110x over previous
import functools

import jax
import jax.numpy as jnp
from jax import lax
from jax.experimental import pallas as pl
from jax.experimental.pallas import tpu as pltpu
from jax.experimental.pallas import tpu_sc as plsc

NUM_CLASSES = 40
OUT_CHANNELS = 512
N = 100000

NUM_WORKERS = 32
CHUNK = 80
NUM_CHUNKS = N // CHUNK
MAX_T = (NUM_CHUNKS + NUM_WORKERS - 1) // NUM_WORKERS
SPAN = MAX_T * CHUNK
N_PAD = (NUM_CHUNKS - 1) * CHUNK + SPAN
LANES = 16
TABLE_ROWS = 48


@functools.partial(
    pl.kernel,
    mesh=plsc.VectorSubcoreMesh(core_axis_name="c", subcore_axis_name="s"),
    out_type=jax.ShapeDtypeStruct((N, OUT_CHANNELS), jnp.float32),
    scratch_types=[
        pltpu.VMEM((TABLE_ROWS, OUT_CHANNELS), jnp.float32),
        pltpu.VMEM((SPAN,), jnp.int32),
        pltpu.VMEM((SPAN,), jnp.int32),
        pltpu.VMEM((SPAN,), jnp.int32),
        pltpu.VMEM((CHUNK, OUT_CHANNELS), jnp.float32),
        pltpu.SemaphoreType.DMA,
        pltpu.SemaphoreType.DMA,
    ],
)
def _masked_gather(y_hbm, m_hbm, emb_hbm, out_hbm,
                   table_v, y_v, m_v, idx_v, dummy_v, sem0, sem1):
    w = lax.axis_index("s") * 2 + lax.axis_index("c")
    nt = jnp.where(w < 2, MAX_T, MAX_T - 1)
    start_chunk = (MAX_T - 1) * w + jnp.minimum(w, 2)
    base = start_chunk * CHUNK

    pltpu.sync_copy(emb_hbm.at[w], table_v)
    pltpu.sync_copy(y_hbm.at[pl.ds(base, SPAN)], y_v)
    pltpu.sync_copy(m_hbm.at[pl.ds(base, SPAN)], m_v)

    def sel_body(i, carry):
        sl = pl.ds(i * LANES, LANES)
        idx_v[sl] = jnp.where(m_v[sl] != 0, y_v[sl], NUM_CLASSES)
        return carry
    lax.fori_loop(0, SPAN // LANES, sel_body, 0)

    sems = (sem0, sem1)

    def issue_chunk(t, b):
        def group_body(g, carry):
            row0 = t * CHUNK + g * LANES
            idxv = idx_v[pl.ds(row0, LANES)]
            for r in range(LANES):
                pltpu.make_async_copy(
                    table_v.at[idxv[r]],
                    out_hbm.at[base + row0 + r],
                    sems[b]).start()
            return carry
        lax.fori_loop(0, CHUNK // LANES, group_body, 0)

    def drain_desc(t, b):
        return pltpu.make_async_copy(
            out_hbm.at[pl.ds(base + t * CHUNK, CHUNK)], dummy_v, sems[b])

    def pair_body(j, carry):
        for h in range(2):
            t = 2 * j + h
            @pl.when(jnp.logical_and(t >= 2, t - 2 < nt))
            def _(t=t, b=h):
                drain_desc(t - 2, b).wait()
            @pl.when(t < nt)
            def _(t=t, b=h):
                issue_chunk(t, b)
        return carry
    lax.fori_loop(0, MAX_T // 2 + 1, pair_body, 0)


def kernel(y, mask, emb):
    y32 = jnp.pad(y.astype(jnp.int32), (0, N_PAD - N))
    m32 = jnp.pad(mask.astype(jnp.int32), (0, N_PAD - N))
    emb2 = jnp.concatenate(
        [emb, jnp.zeros((TABLE_ROWS - NUM_CLASSES, OUT_CHANNELS), emb.dtype)],
        axis=0)
    emb_rep = jnp.broadcast_to(emb2[None], (NUM_WORKERS,) + emb2.shape)
    return _masked_gather(y32, m32, emb_rep)


# speedup vs baseline: 1.9146x; 1.0192x over previous
import functools

import jax
import jax.numpy as jnp
from jax import lax
from jax.experimental import pallas as pl
from jax.experimental.pallas import tpu as pltpu
from jax.experimental.pallas import tpu_sc as plsc

NUM_CLASSES = 40
OUT_CHANNELS = 512
N = 100000

NUM_WORKERS = 32
CHUNK = 80
NUM_CHUNKS = N // CHUNK
MAX_T = (NUM_CHUNKS + NUM_WORKERS - 1) // NUM_WORKERS
SPAN = MAX_T * CHUNK
N_PAD = (NUM_CHUNKS - 1) * CHUNK + SPAN
LANES = 16
TABLE_ROWS = 48


@functools.partial(
    pl.kernel,
    mesh=plsc.VectorSubcoreMesh(core_axis_name="c", subcore_axis_name="s"),
    out_type=jax.ShapeDtypeStruct((N, OUT_CHANNELS), jnp.float32),
    scratch_types=[
        pltpu.VMEM((TABLE_ROWS, OUT_CHANNELS), jnp.float32),
        pltpu.VMEM((SPAN,), jnp.int32),
        pltpu.VMEM((SPAN,), jnp.int32),
        pltpu.VMEM((SPAN,), jnp.int32),
        pltpu.VMEM((CHUNK, OUT_CHANNELS), jnp.float32),
        pltpu.SemaphoreType.DMA,
        pltpu.SemaphoreType.DMA,
        pltpu.SemaphoreType.DMA,
    ],
)
def _masked_gather(y_hbm, m_hbm, emb_hbm, out_hbm,
                   table_v, y_v, m_v, idx_v, dummy_v, sem0, sem1, tsem):
    w = lax.axis_index("s") * 2 + lax.axis_index("c")
    nt = jnp.where(w < 2, MAX_T, MAX_T - 1)
    start_chunk = (MAX_T - 1) * w + jnp.minimum(w, 2)
    base = start_chunk * CHUNK

    table_copy = pltpu.make_async_copy(emb_hbm.at[w], table_v, tsem)
    table_copy.start()
    y_copy = pltpu.make_async_copy(y_hbm.at[pl.ds(base, SPAN)], y_v, sem0)
    m_copy = pltpu.make_async_copy(m_hbm.at[pl.ds(base, SPAN)], m_v, sem1)
    y_copy.start()
    m_copy.start()
    y_copy.wait()
    m_copy.wait()

    def sel_body(i, carry):
        sl = pl.ds(i * LANES, LANES)
        idx_v[sl] = jnp.where(m_v[sl] != 0, y_v[sl], NUM_CLASSES)
        return carry
    lax.fori_loop(0, SPAN // LANES, sel_body, 0)
    table_copy.wait()

    sems = (sem0, sem1)

    def issue_chunk(t, b):
        def group_body(g, carry):
            row0 = t * CHUNK + g * LANES
            idxv = idx_v[pl.ds(row0, LANES)]
            for r in range(LANES):
                pltpu.make_async_copy(
                    table_v.at[idxv[r]],
                    out_hbm.at[base + row0 + r],
                    sems[b]).start()
            return carry
        lax.fori_loop(0, CHUNK // LANES, group_body, 0)

    def drain_desc(t, b):
        return pltpu.make_async_copy(
            out_hbm.at[pl.ds(base + t * CHUNK, CHUNK)], dummy_v, sems[b])

    def pair_body(j, carry):
        for h in range(2):
            t = 2 * j + h
            @pl.when(jnp.logical_and(t >= 2, t - 2 < nt))
            def _(t=t, b=h):
                drain_desc(t - 2, b).wait()
            @pl.when(t < nt)
            def _(t=t, b=h):
                issue_chunk(t, b)
        return carry
    lax.fori_loop(0, MAX_T // 2 + 1, pair_body, 0)


def kernel(y, mask, emb):
    y32 = jnp.pad(y.astype(jnp.int32), (0, N_PAD - N))
    m32 = jnp.pad(mask.astype(jnp.int32), (0, N_PAD - N))
    emb2 = jnp.concatenate(
        [emb, jnp.zeros((TABLE_ROWS - NUM_CLASSES, OUT_CHANNELS), emb.dtype)],
        axis=0)
    emb_rep = jnp.broadcast_to(emb2[None], (NUM_WORKERS,) + emb2.shape)
    return _masked_gather(y32, m32, emb_rep)
